# Initial kernel scaffold; baseline (speedup 1.0000x reference)
import jax
import jax.numpy as jnp
from jax.experimental import pallas as pl


def kernel(x, edge_index, batch, W1, b1, W2, b2, fcW, fcb):
    raise NotImplementedError("write your pallas kernel here")


# speedup vs baseline: 3.0458x; 3.0458x over previous
import functools

import jax
import jax.numpy as jnp
from jax import lax
from jax.experimental import pallas as pl
from jax.experimental.pallas import tpu as pltpu
from jax.experimental.pallas import tpu_sc as plsc

NC = 2
NS = 16
EB = 128
DEGW = 16
G = 64


def _sc_mesh():
    return plsc.VectorSubcoreMesh(
        core_axis_name="c", subcore_axis_name="s", num_cores=NC, num_subcores=NS
    )


def _deg_call(col, n_pad):
    e = col.shape[0]
    nblocks = e // EB
    ntiles = NC * NS
    stripe = n_pad // NS

    @functools.partial(
        pl.kernel,
        out_type=jax.ShapeDtypeStruct((NC, n_pad, DEGW), jnp.float32),
        mesh=_sc_mesh(),
        scratch_types=[
            pltpu.VMEM((EB,), jnp.int32),
            pltpu.VMEM((EB, DEGW), jnp.float32),
            pltpu.VMEM((stripe, DEGW), jnp.float32),
            pltpu.VMEM_SHARED((NS, stripe, DEGW), jnp.float32),
        ],
    )
    def deg_kernel(col_hbm, out_hbm, col_v, ones_v, zero_v, table):
        c = lax.axis_index("c")
        s = lax.axis_index("s")

        def zfill(i, _):
            zero_v[i, :] = jnp.zeros((DEGW,), jnp.float32)
            return 0

        lax.fori_loop(0, stripe, zfill, 0)
        pltpu.sync_copy(zero_v, table.at[s])

        def ofill(i, _):
            ones_v[i, :] = jnp.full((DEGW,), 1.0, jnp.float32)
            return 0

        lax.fori_loop(0, EB, ofill, 0)
        pltpu.sync_copy(table.at[s, pl.ds(0, EB)], ones_v)
        pltpu.sync_copy(
            ones_v,
            out_hbm.at[c, pl.ds(s * stripe, EB)],
        )
        pltpu.sync_copy(
            zero_v.at[pl.ds(EB, stripe - EB)],
            out_hbm.at[c, pl.ds(s * stripe + EB, stripe - EB)],
        )

    return deg_kernel(col)


def _scatter_call(g, row, col):
    n_pad, d = g.shape
    e = row.shape[0]
    nblocks = e // EB
    ntiles = NC * NS
    stripe = n_pad // NS
    zrows = 128

    @functools.partial(
        pl.kernel,
        out_type=jax.ShapeDtypeStruct((NC, n_pad, d), jnp.float32),
        mesh=_sc_mesh(),
        scratch_types=[
            pltpu.VMEM((EB,), jnp.int32),
            pltpu.VMEM((EB,), jnp.int32),
            pltpu.VMEM((EB, d), jnp.float32),
            pltpu.VMEM((zrows, d), jnp.float32),
            pltpu.VMEM_SHARED((n_pad, d), jnp.float32),
            pltpu.SemaphoreType.DMA,
        ],
    )
    def scatter_kernel(g_hbm, row_hbm, col_hbm, out_hbm,
                       rowi_v, coli_v, rows_v, zero_v, acc, sem):
        c = lax.axis_index("c")
        s = lax.axis_index("s")
        wid = s * NC + c

        def zfill(i, _):
            for j in range(d // 16):
                zero_v[i, pl.ds(j * 16, 16)] = jnp.zeros((16,), jnp.float32)
            return 0

        lax.fori_loop(0, zrows, zfill, 0)
        for k in range(stripe // zrows):
            pltpu.sync_copy(zero_v, acc.at[pl.ds(s * stripe + k * zrows, zrows)])
        plsc.subcore_barrier()

        nb = (nblocks - wid + ntiles - 1) // ntiles

        def step(m, _):
            base = (wid + m * ntiles) * EB
            pltpu.sync_copy(row_hbm.at[pl.ds(base, EB)], rowi_v)
            cp = pltpu.async_copy(g_hbm.at[rowi_v], rows_v, sem)
            pltpu.sync_copy(col_hbm.at[pl.ds(base, EB)], coli_v)
            cp.wait()
            pltpu.sync_copy(rows_v, acc.at[coli_v], add=True)
            return 0

        lax.fori_loop(0, nb, step, 0)
        plsc.subcore_barrier()
        pltpu.sync_copy(
            acc.at[pl.ds(s * stripe, stripe)],
            out_hbm.at[c, pl.ds(s * stripe, stripe)],
        )

    return scatter_kernel(g, row, col)


_BLK = 512


def _mm1_call(x_p, w1, deg_a, deg_b):
    n_pad, d = x_p.shape
    grid = n_pad // _BLK

    def body(x_ref, w_ref, da_ref, db_ref, g1_ref, dis_ref):
        deg = da_ref[:, :1] + db_ref[:, :1] + 1.0
        dis = lax.rsqrt(deg)
        h = jnp.dot(x_ref[...], w_ref[...], preferred_element_type=jnp.float32)
        g1_ref[...] = h * dis
        dis_ref[...] = dis

    return pl.pallas_call(
        body,
        grid=(grid,),
        in_specs=[
            pl.BlockSpec((_BLK, d), lambda i: (i, 0)),
            pl.BlockSpec((d, d), lambda i: (0, 0)),
            pl.BlockSpec((_BLK, DEGW), lambda i: (i, 0)),
            pl.BlockSpec((_BLK, DEGW), lambda i: (i, 0)),
        ],
        out_specs=[
            pl.BlockSpec((_BLK, d), lambda i: (i, 0)),
            pl.BlockSpec((_BLK, 1), lambda i: (i, 0)),
        ],
        out_shape=[
            jax.ShapeDtypeStruct((n_pad, d), jnp.float32),
            jax.ShapeDtypeStruct((n_pad, 1), jnp.float32),
        ],
    )(x_p, w1, deg_a, deg_b)


def _mm2_call(g1, acc_a, acc_b, dis_col, w2, b1):
    n_pad, d = g1.shape
    grid = n_pad // _BLK

    def body(g1_ref, aa_ref, ab_ref, dis_ref, w_ref, b_ref, g2_ref):
        a = g1_ref[...] + aa_ref[...] + ab_ref[...]
        t = jnp.maximum(dis_ref[...] * a + b_ref[...], 0.0)
        g2_ref[...] = jnp.dot(
            t, w_ref[...], preferred_element_type=jnp.float32
        ) * dis_ref[...]

    return pl.pallas_call(
        body,
        grid=(grid,),
        in_specs=[
            pl.BlockSpec((_BLK, d), lambda i: (i, 0)),
            pl.BlockSpec((_BLK, d), lambda i: (i, 0)),
            pl.BlockSpec((_BLK, d), lambda i: (i, 0)),
            pl.BlockSpec((_BLK, 1), lambda i: (i, 0)),
            pl.BlockSpec((d, d), lambda i: (0, 0)),
            pl.BlockSpec((1, d), lambda i: (0, 0)),
        ],
        out_specs=pl.BlockSpec((_BLK, d), lambda i: (i, 0)),
        out_shape=jax.ShapeDtypeStruct((n_pad, d), jnp.float32),
    )(g1, acc_a, acc_b, dis_col, w2, b1)


def _final_call(g2, acc_a, acc_b, dis_col, batch_col, b2, fcw, fcb):
    n_pad, d = g2.shape
    dout = fcw.shape[1]
    grid = n_pad // _BLK

    def body(g2_ref, aa_ref, ab_ref, dis_ref, bt_ref, b_ref, fcw_ref, fcb_ref,
             out_ref, sums_ref, cnt_ref):
        i = pl.program_id(0)

        @pl.when(i == 0)
        def _():
            sums_ref[...] = jnp.zeros_like(sums_ref)
            cnt_ref[...] = jnp.zeros_like(cnt_ref)

        a = g2_ref[...] + aa_ref[...] + ab_ref[...]
        r = jnp.maximum(dis_ref[...] * a + b_ref[...], 0.0)
        bt = bt_ref[...]
        p = (bt == lax.broadcasted_iota(jnp.int32, (_BLK, G), 1)).astype(
            jnp.float32
        )
        dn = (((0,), (0,)), ((), ()))
        sums_ref[...] += lax.dot_general(
            p, r, dn, preferred_element_type=jnp.float32
        )
        cnt_ref[...] += lax.dot_general(
            p, jnp.ones((_BLK, 1), jnp.float32), dn,
            preferred_element_type=jnp.float32,
        )

        @pl.when(i == grid - 1)
        def _():
            pooled = sums_ref[...] / jnp.maximum(cnt_ref[...], 1.0)
            out_ref[...] = jnp.dot(
                pooled, fcw_ref[...], preferred_element_type=jnp.float32
            ) + fcb_ref[...]

    return pl.pallas_call(
        body,
        grid=(grid,),
        in_specs=[
            pl.BlockSpec((_BLK, d), lambda i: (i, 0)),
            pl.BlockSpec((_BLK, d), lambda i: (i, 0)),
            pl.BlockSpec((_BLK, d), lambda i: (i, 0)),
            pl.BlockSpec((_BLK, 1), lambda i: (i, 0)),
            pl.BlockSpec((_BLK, 1), lambda i: (i, 0)),
            pl.BlockSpec((1, d), lambda i: (0, 0)),
            pl.BlockSpec((d, dout), lambda i: (0, 0)),
            pl.BlockSpec((1, dout), lambda i: (0, 0)),
        ],
        out_specs=pl.BlockSpec((G, dout), lambda i: (0, 0)),
        out_shape=jax.ShapeDtypeStruct((G, dout), jnp.float32),
        scratch_shapes=[
            pltpu.VMEM((G, d), jnp.float32),
            pltpu.VMEM((G, 1), jnp.float32),
        ],
    )(g2, acc_a, acc_b, dis_col, batch_col, b2, fcw, fcb)


def kernel(x, edge_index, batch, W1, b1, W2, b2, fcW, fcb):
    n, d = x.shape
    n_pad = ((n + 2047) // 2048) * 2048

    row = edge_index[0].astype(jnp.int32)
    col = edge_index[1].astype(jnp.int32)
    x_p = jnp.pad(x, ((0, n_pad - n), (0, 0)))
    batch_col = jnp.pad(batch, (0, n_pad - n), constant_values=G).astype(
        jnp.int32
    )[:, None]

    deg_flat = jax.ops.segment_sum(
        jnp.ones((col.shape[0],), jnp.float32), col, num_segments=n_pad
    )
    deg_a = jnp.broadcast_to(deg_flat[:, None], (n_pad, DEGW))
    deg_b = jnp.zeros((n_pad, DEGW), jnp.float32)

    def _jscatter(gv):
        a = jax.ops.segment_sum(gv[row], col, num_segments=n_pad)
        return a, jnp.zeros_like(a)

    g1, dis_col = _mm1_call(x_p, W1, deg_a, deg_b)
    acc1a, acc1b = _jscatter(g1)
    g2 = _mm2_call(g1, acc1a, acc1b, dis_col, W2, b1[None, :])
    acc2a, acc2b = _jscatter(g2)
    return _final_call(
        g2, acc2a, acc2b, dis_col, batch_col, b2[None, :], fcW, fcb[None, :]
    )


# speedup vs baseline: 4.0845x; 1.3410x over previous
import functools

import jax
import jax.numpy as jnp
from jax import lax
from jax.experimental import pallas as pl
from jax.experimental.pallas import tpu as pltpu
from jax.experimental.pallas import tpu_sc as plsc

NC = 2
NS = 16
NT = NC * NS
CG = 4
IB = 128
NSUB = 10
CB = IB * NSUB
G = 64


def _sc_mesh():
    return plsc.VectorSubcoreMesh(
        core_axis_name="c", subcore_axis_name="s", num_cores=NC, num_subcores=NS
    )


def _deg_call(col, n_pad):
    e = col.shape[0]
    epw = e // NT
    dcb = 2000
    nch = epw // dcb
    nv = n_pad // 16

    @functools.partial(
        pl.kernel,
        out_type=jax.ShapeDtypeStruct((NT, n_pad), jnp.float32),
        mesh=_sc_mesh(),
        compiler_params=pltpu.CompilerParams(needs_layout_passes=False),
        scratch_types=[
            pltpu.VMEM((dcb,), jnp.int32),
            pltpu.VMEM((n_pad,), jnp.float32),
        ],
    )
    def deg_kernel(col_hbm, out_hbm, colb_v, table_v):
        c = lax.axis_index("c")
        s = lax.axis_index("s")
        wid = s * NC + c

        def zfill(i, _):
            table_v[pl.ds(i * 16, 16)] = jnp.zeros((16,), jnp.float32)
            return 0

        lax.fori_loop(0, nv, zfill, 0)

        ones16 = jnp.full((16,), 1.0, jnp.float32)

        def chunk(k, _):
            base = wid * epw + k * dcb
            pltpu.sync_copy(col_hbm.at[pl.ds(base, dcb)], colb_v)

            def grp(j, _):
                idx = colb_v[pl.ds(j * 16, 16)]
                plsc.addupdate_scatter(table_v, [idx], ones16)
                return 0

            lax.fori_loop(0, dcb // 16, grp, 0)
            return 0

        lax.fori_loop(0, nch, chunk, 0)
        pltpu.sync_copy(table_v, out_hbm.at[wid])

    return deg_kernel(col)


def _scatter_call(gf, row, col, n_pad):
    e = row.shape[0]
    scb = 2000
    nch = e // scb
    accn = n_pad * CG

    @functools.partial(
        pl.kernel,
        out_type=jax.ShapeDtypeStruct((NT, accn), jnp.float32),
        mesh=_sc_mesh(),
        compiler_params=pltpu.CompilerParams(needs_layout_passes=False),
        scratch_types=[
            pltpu.VMEM((scb,), jnp.int32),
            pltpu.VMEM((scb,), jnp.int32),
            pltpu.VMEM((accn,), jnp.float32),
            pltpu.VMEM((accn,), jnp.float32),
        ],
    )
    def scatter_kernel(gf_hbm, row_hbm, col_hbm, out_hbm,
                       rowb_v, colb_v, gtab_v, acc_v):
        c = lax.axis_index("c")
        s = lax.axis_index("s")
        wid = s * NC + c

        pltpu.sync_copy(gf_hbm.at[pl.ds(wid * accn, accn)], gtab_v)

        def zfill(i, _):
            acc_v[pl.ds(i * 16, 16)] = jnp.zeros((16,), jnp.float32)
            return 0

        lax.fori_loop(0, accn // 16, zfill, 0)

        def chunk(k, _):
            pltpu.sync_copy(row_hbm.at[pl.ds(k * scb, scb)], rowb_v)
            pltpu.sync_copy(col_hbm.at[pl.ds(k * scb, scb)], colb_v)

            def grp(j, _):
                rowv = rowb_v[pl.ds(j * 16, 16)]
                colv = colb_v[pl.ds(j * 16, 16)]
                row4 = rowv * CG
                col4 = colv * CG
                for ch in range(CG):
                    vals = plsc.load_gather(gtab_v, [row4 + ch])
                    plsc.addupdate_scatter(acc_v, [col4 + ch], vals)
                return 0

            lax.fori_loop(0, scb // 16, grp, 0)
            return 0

        lax.fori_loop(0, nch, chunk, 0)
        pltpu.sync_copy(acc_v, out_hbm.at[wid])

    return scatter_kernel(gf, row, col)


_BLK = 512


def _mm1_call(x_p, w1, degp):
    n_pad, d = x_p.shape
    grid = n_pad // _BLK

    def body(x_ref, w_ref, dp_ref, g1_ref, dis_ref):
        ones = jnp.ones((NT, 1), jnp.float32)
        dn = (((0,), (0,)), ((), ()))
        deg = lax.dot_general(
            dp_ref[...], ones, dn, preferred_element_type=jnp.float32
        ) + 1.0
        dis = lax.rsqrt(deg)
        h = jnp.dot(x_ref[...], w_ref[...], preferred_element_type=jnp.float32)
        g1_ref[...] = h * dis
        dis_ref[...] = dis

    return pl.pallas_call(
        body,
        grid=(grid,),
        in_specs=[
            pl.BlockSpec((_BLK, d), lambda i: (i, 0)),
            pl.BlockSpec((d, d), lambda i: (0, 0)),
            pl.BlockSpec((NT, _BLK), lambda i: (0, i)),
        ],
        out_specs=[
            pl.BlockSpec((_BLK, d), lambda i: (i, 0)),
            pl.BlockSpec((_BLK, 1), lambda i: (i, 0)),
        ],
        out_shape=[
            jax.ShapeDtypeStruct((n_pad, d), jnp.float32),
            jax.ShapeDtypeStruct((n_pad, 1), jnp.float32),
        ],
    )(x_p, w1, degp)


def _mm2_call(g1, acc1, dis_col, w2, b1):
    n_pad, d = g1.shape
    grid = n_pad // _BLK

    def body(g1_ref, a_ref, dis_ref, w_ref, b_ref, g2_ref):
        a = g1_ref[...] + a_ref[...]
        t = jnp.maximum(dis_ref[...] * a + b_ref[...], 0.0)
        g2_ref[...] = jnp.dot(
            t, w_ref[...], preferred_element_type=jnp.float32
        ) * dis_ref[...]

    return pl.pallas_call(
        body,
        grid=(grid,),
        in_specs=[
            pl.BlockSpec((_BLK, d), lambda i: (i, 0)),
            pl.BlockSpec((_BLK, d), lambda i: (i, 0)),
            pl.BlockSpec((_BLK, 1), lambda i: (i, 0)),
            pl.BlockSpec((d, d), lambda i: (0, 0)),
            pl.BlockSpec((1, d), lambda i: (0, 0)),
        ],
        out_specs=pl.BlockSpec((_BLK, d), lambda i: (i, 0)),
        out_shape=jax.ShapeDtypeStruct((n_pad, d), jnp.float32),
    )(g1, acc1, dis_col, w2, b1)


def _final_call(g2, acc2, dis_col, batch_col, b2, fcw, fcb):
    n_pad, d = g2.shape
    dout = fcw.shape[1]
    grid = n_pad // _BLK

    def body(g2_ref, a_ref, dis_ref, bt_ref, b_ref, fcw_ref, fcb_ref,
             out_ref, sums_ref, cnt_ref):
        i = pl.program_id(0)

        @pl.when(i == 0)
        def _():
            sums_ref[...] = jnp.zeros_like(sums_ref)
            cnt_ref[...] = jnp.zeros_like(cnt_ref)

        a = g2_ref[...] + a_ref[...]
        r = jnp.maximum(dis_ref[...] * a + b_ref[...], 0.0)
        bt = bt_ref[...]
        p = (bt == lax.broadcasted_iota(jnp.int32, (_BLK, G), 1)).astype(
            jnp.float32
        )
        dn = (((0,), (0,)), ((), ()))
        sums_ref[...] += lax.dot_general(
            p, r, dn, preferred_element_type=jnp.float32
        )
        cnt_ref[...] += lax.dot_general(
            p, jnp.ones((_BLK, 1), jnp.float32), dn,
            preferred_element_type=jnp.float32,
        )

        @pl.when(i == grid - 1)
        def _():
            pooled = sums_ref[...] / jnp.maximum(cnt_ref[...], 1.0)
            out_ref[...] = jnp.dot(
                pooled, fcw_ref[...], preferred_element_type=jnp.float32
            ) + fcb_ref[...]

    return pl.pallas_call(
        body,
        grid=(grid,),
        in_specs=[
            pl.BlockSpec((_BLK, d), lambda i: (i, 0)),
            pl.BlockSpec((_BLK, d), lambda i: (i, 0)),
            pl.BlockSpec((_BLK, 1), lambda i: (i, 0)),
            pl.BlockSpec((_BLK, 1), lambda i: (i, 0)),
            pl.BlockSpec((1, d), lambda i: (0, 0)),
            pl.BlockSpec((d, dout), lambda i: (0, 0)),
            pl.BlockSpec((1, dout), lambda i: (0, 0)),
        ],
        out_specs=pl.BlockSpec((G, dout), lambda i: (0, 0)),
        out_shape=jax.ShapeDtypeStruct((G, dout), jnp.float32),
        scratch_shapes=[
            pltpu.VMEM((G, d), jnp.float32),
            pltpu.VMEM((G, 1), jnp.float32),
        ],
    )(g2, acc2, dis_col, batch_col, b2, fcw, fcb)


def _to_groups(g, n_pad):
    return g.reshape(n_pad, NT, CG).transpose(1, 0, 2).reshape(NT * n_pad * CG)


def _from_groups(acc, n_pad):
    return acc.reshape(NT, n_pad, CG).transpose(1, 0, 2).reshape(n_pad, NT * CG)


def kernel(x, edge_index, batch, W1, b1, W2, b2, fcW, fcb):
    n, d = x.shape
    n_pad = ((n + 2047) // 2048) * 2048

    row = edge_index[0].astype(jnp.int32)
    col = edge_index[1].astype(jnp.int32)
    x_p = jnp.pad(x, ((0, n_pad - n), (0, 0)))
    batch_col = jnp.pad(batch, (0, n_pad - n), constant_values=G).astype(
        jnp.int32
    )[:, None]

    degp = _deg_call(col, n_pad)
    g1, dis_col = _mm1_call(x_p, W1, degp)
    acc1 = _scatter_call(_to_groups(g1, n_pad), row, col, n_pad)
    g2 = _mm2_call(g1, _from_groups(acc1, n_pad), dis_col, W2, b1[None, :])
    acc2 = _scatter_call(_to_groups(g2, n_pad), row, col, n_pad)
    return _final_call(
        g2, _from_groups(acc2, n_pad), dis_col, batch_col, b2[None, :],
        fcW, fcb[None, :],
    )


# speedup vs baseline: 7.5744x; 1.8544x over previous
import functools

import jax
import jax.numpy as jnp
from jax import lax
from jax.experimental import pallas as pl
from jax.experimental.pallas import tpu as pltpu
from jax.experimental.pallas import tpu_sc as plsc

NC = 2
NS = 16
NT = NC * NS
CG = 4
IB = 128
NSUB = 10
CB = IB * NSUB
G = 64


def _sc_mesh():
    return plsc.VectorSubcoreMesh(
        core_axis_name="c", subcore_axis_name="s", num_cores=NC, num_subcores=NS
    )


def _deg_call(col, n_pad):
    e = col.shape[0]
    epw = e // NT
    dcb = 10000
    nch = epw // dcb
    nv = n_pad // 16

    @functools.partial(
        pl.kernel,
        out_type=jax.ShapeDtypeStruct((NT, n_pad), jnp.float32),
        mesh=_sc_mesh(),
        compiler_params=pltpu.CompilerParams(needs_layout_passes=False),
        scratch_types=[
            pltpu.VMEM((dcb,), jnp.int32),
            pltpu.VMEM((n_pad,), jnp.float32),
        ],
    )
    def deg_kernel(col_hbm, out_hbm, colb_v, table_v):
        c = lax.axis_index("c")
        s = lax.axis_index("s")
        wid = s * NC + c

        def zfill(i, _):
            table_v[pl.ds(i * 16, 16)] = jnp.zeros((16,), jnp.float32)
            return 0

        lax.fori_loop(0, nv, zfill, 0)

        ones16 = jnp.full((16,), 1.0, jnp.float32)

        def chunk(k, _):
            base = wid * epw + k * dcb
            pltpu.sync_copy(col_hbm.at[pl.ds(base, dcb)], colb_v)

            @plsc.parallel_loop(0, dcb // 16, unroll=4)
            def grp(j):
                idx = colb_v[pl.ds(j * 16, 16)]
                plsc.addupdate_scatter(table_v, [idx], ones16)

            return 0

        lax.fori_loop(0, nch, chunk, 0)
        pltpu.sync_copy(table_v, out_hbm.at[wid])

    return deg_kernel(col)


def _scatter_call(gf, row, col, n_pad):
    e = row.shape[0]
    scb = 10000
    nch = e // scb
    accn = n_pad * CG

    @functools.partial(
        pl.kernel,
        out_type=jax.ShapeDtypeStruct((NT, accn), jnp.float32),
        mesh=_sc_mesh(),
        compiler_params=pltpu.CompilerParams(needs_layout_passes=False),
        scratch_types=[
            pltpu.VMEM((scb,), jnp.int32),
            pltpu.VMEM((scb,), jnp.int32),
            pltpu.VMEM((accn,), jnp.float32),
            pltpu.VMEM((accn,), jnp.float32),
        ],
    )
    def scatter_kernel(gf_hbm, row_hbm, col_hbm, out_hbm,
                       rowb_v, colb_v, gtab_v, acc_v):
        c = lax.axis_index("c")
        s = lax.axis_index("s")
        wid = s * NC + c

        pltpu.sync_copy(gf_hbm.at[pl.ds(wid * accn, accn)], gtab_v)

        def zfill(i, _):
            acc_v[pl.ds(i * 16, 16)] = jnp.zeros((16,), jnp.float32)
            return 0

        lax.fori_loop(0, accn // 16, zfill, 0)

        def chunk(k, _):
            pltpu.sync_copy(row_hbm.at[pl.ds(k * scb, scb)], rowb_v)
            pltpu.sync_copy(col_hbm.at[pl.ds(k * scb, scb)], colb_v)

            @plsc.parallel_loop(0, scb // 16, unroll=2)
            def grp(j):
                rowv = rowb_v[pl.ds(j * 16, 16)]
                colv = colb_v[pl.ds(j * 16, 16)]
                row4 = rowv * CG
                col4 = colv * CG
                for ch in range(CG):
                    vals = plsc.load_gather(gtab_v, [row4 + ch])
                    plsc.addupdate_scatter(acc_v, [col4 + ch], vals)

            return 0

        lax.fori_loop(0, nch, chunk, 0)
        pltpu.sync_copy(acc_v, out_hbm.at[wid])

    return scatter_kernel(gf, row, col)


_BLK = 512


def _mm1_call(x_p, w1, degp):
    n_pad, d = x_p.shape
    grid = n_pad // _BLK

    def body(x_ref, w_ref, dp_ref, g1_ref, dis_ref):
        ones = jnp.ones((NT, 1), jnp.float32)
        dn = (((0,), (0,)), ((), ()))
        deg = lax.dot_general(
            dp_ref[...], ones, dn, preferred_element_type=jnp.float32
        ) + 1.0
        dis = lax.rsqrt(deg)
        h = jnp.dot(x_ref[...], w_ref[...], preferred_element_type=jnp.float32)
        g1_ref[...] = h * dis
        dis_ref[...] = dis

    return pl.pallas_call(
        body,
        grid=(grid,),
        in_specs=[
            pl.BlockSpec((_BLK, d), lambda i: (i, 0)),
            pl.BlockSpec((d, d), lambda i: (0, 0)),
            pl.BlockSpec((NT, _BLK), lambda i: (0, i)),
        ],
        out_specs=[
            pl.BlockSpec((_BLK, d), lambda i: (i, 0)),
            pl.BlockSpec((_BLK, 1), lambda i: (i, 0)),
        ],
        out_shape=[
            jax.ShapeDtypeStruct((n_pad, d), jnp.float32),
            jax.ShapeDtypeStruct((n_pad, 1), jnp.float32),
        ],
    )(x_p, w1, degp)


def _mm2_call(g1, acc1, dis_col, w2, b1):
    n_pad, d = g1.shape
    grid = n_pad // _BLK

    def body(g1_ref, a_ref, dis_ref, w_ref, b_ref, g2_ref):
        a = g1_ref[...] + a_ref[...]
        t = jnp.maximum(dis_ref[...] * a + b_ref[...], 0.0)
        g2_ref[...] = jnp.dot(
            t, w_ref[...], preferred_element_type=jnp.float32
        ) * dis_ref[...]

    return pl.pallas_call(
        body,
        grid=(grid,),
        in_specs=[
            pl.BlockSpec((_BLK, d), lambda i: (i, 0)),
            pl.BlockSpec((_BLK, d), lambda i: (i, 0)),
            pl.BlockSpec((_BLK, 1), lambda i: (i, 0)),
            pl.BlockSpec((d, d), lambda i: (0, 0)),
            pl.BlockSpec((1, d), lambda i: (0, 0)),
        ],
        out_specs=pl.BlockSpec((_BLK, d), lambda i: (i, 0)),
        out_shape=jax.ShapeDtypeStruct((n_pad, d), jnp.float32),
    )(g1, acc1, dis_col, w2, b1)


def _final_call(g2, acc2, dis_col, batch_col, b2, fcw, fcb):
    n_pad, d = g2.shape
    dout = fcw.shape[1]
    grid = n_pad // _BLK

    def body(g2_ref, a_ref, dis_ref, bt_ref, b_ref, fcw_ref, fcb_ref,
             out_ref, sums_ref, cnt_ref):
        i = pl.program_id(0)

        @pl.when(i == 0)
        def _():
            sums_ref[...] = jnp.zeros_like(sums_ref)
            cnt_ref[...] = jnp.zeros_like(cnt_ref)

        a = g2_ref[...] + a_ref[...]
        r = jnp.maximum(dis_ref[...] * a + b_ref[...], 0.0)
        bt = bt_ref[...]
        p = (bt == lax.broadcasted_iota(jnp.int32, (_BLK, G), 1)).astype(
            jnp.float32
        )
        dn = (((0,), (0,)), ((), ()))
        sums_ref[...] += lax.dot_general(
            p, r, dn, preferred_element_type=jnp.float32
        )
        cnt_ref[...] += lax.dot_general(
            p, jnp.ones((_BLK, 1), jnp.float32), dn,
            preferred_element_type=jnp.float32,
        )

        @pl.when(i == grid - 1)
        def _():
            pooled = sums_ref[...] / jnp.maximum(cnt_ref[...], 1.0)
            out_ref[...] = jnp.dot(
                pooled, fcw_ref[...], preferred_element_type=jnp.float32
            ) + fcb_ref[...]

    return pl.pallas_call(
        body,
        grid=(grid,),
        in_specs=[
            pl.BlockSpec((_BLK, d), lambda i: (i, 0)),
            pl.BlockSpec((_BLK, d), lambda i: (i, 0)),
            pl.BlockSpec((_BLK, 1), lambda i: (i, 0)),
            pl.BlockSpec((_BLK, 1), lambda i: (i, 0)),
            pl.BlockSpec((1, d), lambda i: (0, 0)),
            pl.BlockSpec((d, dout), lambda i: (0, 0)),
            pl.BlockSpec((1, dout), lambda i: (0, 0)),
        ],
        out_specs=pl.BlockSpec((G, dout), lambda i: (0, 0)),
        out_shape=jax.ShapeDtypeStruct((G, dout), jnp.float32),
        scratch_shapes=[
            pltpu.VMEM((G, d), jnp.float32),
            pltpu.VMEM((G, 1), jnp.float32),
        ],
    )(g2, acc2, dis_col, batch_col, b2, fcw, fcb)


def _to_groups(g, n_pad):
    return g.reshape(n_pad, NT, CG).transpose(1, 0, 2).reshape(NT * n_pad * CG)


def _from_groups(acc, n_pad):
    return acc.reshape(NT, n_pad, CG).transpose(1, 0, 2).reshape(n_pad, NT * CG)


def kernel(x, edge_index, batch, W1, b1, W2, b2, fcW, fcb):
    n, d = x.shape
    n_pad = ((n + 2047) // 2048) * 2048

    row = edge_index[0].astype(jnp.int32)
    col = edge_index[1].astype(jnp.int32)
    x_p = jnp.pad(x, ((0, n_pad - n), (0, 0)))
    batch_col = jnp.pad(batch, (0, n_pad - n), constant_values=G).astype(
        jnp.int32
    )[:, None]

    degp = _deg_call(col, n_pad)
    g1, dis_col = _mm1_call(x_p, W1, degp)
    acc1 = _scatter_call(_to_groups(g1, n_pad), row, col, n_pad)
    g2 = _mm2_call(g1, _from_groups(acc1, n_pad), dis_col, W2, b1[None, :])
    acc2 = _scatter_call(_to_groups(g2, n_pad), row, col, n_pad)
    return _final_call(
        g2, _from_groups(acc2, n_pad), dis_col, batch_col, b2[None, :],
        fcW, fcb[None, :],
    )
